# baseline (speedup 1.0000x reference)
import functools

import jax
import jax.numpy as jnp
from jax.experimental import pallas as pl
from jax.experimental.pallas import tpu as pltpu

_BATCH_BLOCK = 1024
_VOCAB_BLOCK = 2048


def _scale_kernel(temp_ref, gid_ref, x_ref, o_ref):
    g = gid_ref[:]
    num_groups = temp_ref.shape[0]
    s = jnp.zeros(g.shape, dtype=jnp.float32)
    for gid in range(num_groups):
        s = jnp.where(g == gid, 1.0 / temp_ref[gid], s)
    o_ref[...] = x_ref[...] * s[:, None]


def kernel(logits, group_ids, temperatures):
    batch, vocab = logits.shape
    bm = min(_BATCH_BLOCK, batch)
    bn = _VOCAB_BLOCK
    grid = (pl.cdiv(batch, bm), pl.cdiv(vocab, bn))
    return pl.pallas_call(
        _scale_kernel,
        grid=grid,
        in_specs=[
            pl.BlockSpec(memory_space=pltpu.SMEM),
            pl.BlockSpec((bm,), lambda i, j: (i,)),
            pl.BlockSpec((bm, bn), lambda i, j: (i, j)),
        ],
        out_specs=pl.BlockSpec((bm, bn), lambda i, j: (i, j)),
        out_shape=jax.ShapeDtypeStruct((batch, vocab), logits.dtype),
    )(temperatures, group_ids, logits)
